# Initial kernel scaffold; baseline (speedup 1.0000x reference)
import jax
import jax.numpy as jnp
from jax.experimental import pallas as pl


def kernel(x, edge_index, W1_0, b1_0, W2_0, b2_0, W1_1, b1_1, W2_1, b2_1, W1_2, b1_2, W2_2, b2_2):
    raise NotImplementedError("write your pallas kernel here")


# speedup vs baseline: 3.1776x; 3.1776x over previous
import functools

import jax
import jax.numpy as jnp
from jax import lax
from jax.experimental import pallas as pl
from jax.experimental.pallas import tpu as pltpu
from jax.experimental.pallas import tpu_sc as plsc

N = 10000
E = 320000
D = 128

NC = 2
NS = 16
NW = NC * NS

CHUNK = 128
CPW = 80
EPW = CPW * CHUNK
E_PAD = EPW * NW

ROWS_PER_TILE = 640
N_SP = ROWS_PER_TILE * NS

_sc_mesh = plsc.VectorSubcoreMesh(core_axis_name="c", subcore_axis_name="s")


@functools.partial(
    pl.kernel,
    out_type=jax.ShapeDtypeStruct((NC, N_SP, D), jnp.float32),
    mesh=_sc_mesh,
    scratch_types=[
        pltpu.VMEM((4, CHUNK), jnp.int32),
        pltpu.VMEM((4, CHUNK), jnp.int32),
        pltpu.VMEM((2, CHUNK, D), jnp.float32),
        pltpu.VMEM_SHARED((N_SP, D), jnp.float32),
        pltpu.SemaphoreType.DMA,
        pltpu.SemaphoreType.DMA,
        [pltpu.SemaphoreType.DMA] * 4,
    ],
)
def _sc_agg(h_hbm, src_hbm, dst_hbm, zeros_hbm, out_hbm,
            src_v, dst_v, rows_v, agg_sh, gsem0, gsem1, isems):
    cid = lax.axis_index("c")
    sid = lax.axis_index("s")
    wid = cid * NS + sid
    gsems = (gsem0, gsem1)

    row0 = sid * ROWS_PER_TILE
    pltpu.sync_copy(zeros_hbm.at[pl.ds(row0, ROWS_PER_TILE)],
                    agg_sh.at[pl.ds(row0, ROWS_PER_TILE)])
    pltpu.sync_copy(src_hbm.at[wid, 0], src_v.at[0])
    pltpu.sync_copy(dst_hbm.at[wid, 0], dst_v.at[0])
    plsc.subcore_barrier()
    pltpu.async_copy(h_hbm.at[src_v.at[0]], rows_v.at[0], gsem0)
    for s in (1, 2, 3):
        pltpu.async_copy(src_hbm.at[wid, s], src_v.at[s], isems[s])
        pltpu.async_copy(dst_hbm.at[wid, s], dst_v.at[s], isems[s])

    def _step(j, u):
        p, s = u % 2, u % 4
        s1 = (u + 1) % 4

        @pl.when(j + 1 < CPW)
        def _():
            pltpu.make_async_copy(src_hbm.at[wid, j + 1], src_v.at[s1],
                                  isems[s1]).wait()
            pltpu.make_async_copy(dst_hbm.at[wid, j + 1], dst_v.at[s1],
                                  isems[s1]).wait()
            pltpu.async_copy(h_hbm.at[src_v.at[s1]], rows_v.at[1 - p],
                             gsems[1 - p])

        pltpu.make_async_copy(h_hbm.at[src_v.at[s]], rows_v.at[p],
                              gsems[p]).wait()
        pltpu.sync_copy(rows_v.at[p], agg_sh.at[dst_v.at[s]], add=True)

        @pl.when(j + 4 < CPW)
        def _():
            pltpu.async_copy(src_hbm.at[wid, j + 4], src_v.at[s], isems[s])
            pltpu.async_copy(dst_hbm.at[wid, j + 4], dst_v.at[s], isems[s])

    def body(i, carry):
        for u in range(4):
            _step(4 * i + u, u)
        return carry

    lax.fori_loop(0, CPW // 4, body, 0)
    plsc.subcore_barrier()

    pltpu.sync_copy(agg_sh.at[pl.ds(row0, ROWS_PER_TILE)],
                    out_hbm.at[cid, pl.ds(row0, ROWS_PER_TILE)])


ROW_BLK = 1000


def _tc_mlp_body(h_ref, a0_ref, a1_ref, w1_ref, b1_ref, w2_ref, b2_ref, out_ref):
    m = h_ref[...] + a0_ref[0] + a1_ref[0]
    t = jnp.dot(m, w1_ref[...], preferred_element_type=jnp.float32) + b1_ref[...]
    t = jnp.maximum(t, 0.0)
    u = jnp.dot(t, w2_ref[...], preferred_element_type=jnp.float32) + b2_ref[...]
    out_ref[...] = jnp.where(u > 0.0, u, jnp.exp(jnp.minimum(u, 0.0)) - 1.0)


def _tc_mlp(h, agg, w1, b1, w2, b2):
    grid = N // ROW_BLK
    return pl.pallas_call(
        _tc_mlp_body,
        grid=(grid,),
        in_specs=[
            pl.BlockSpec((ROW_BLK, D), lambda i: (i, 0)),
            pl.BlockSpec((1, ROW_BLK, D), lambda i: (0, i, 0)),
            pl.BlockSpec((1, ROW_BLK, D), lambda i: (1, i, 0)),
            pl.BlockSpec((D, D), lambda i: (0, 0)),
            pl.BlockSpec((1, D), lambda i: (0, 0)),
            pl.BlockSpec((D, D), lambda i: (0, 0)),
            pl.BlockSpec((1, D), lambda i: (0, 0)),
        ],
        out_specs=pl.BlockSpec((ROW_BLK, D), lambda i: (i, 0)),
        out_shape=jax.ShapeDtypeStruct((N, D), jnp.float32),
    )(h, agg, agg, w1, b1, w2, b2)


def kernel(x, edge_index, W1_0, b1_0, W2_0, b2_0, W1_1, b1_1, W2_1, b2_1,
           W1_2, b1_2, W2_2, b2_2):
    src = edge_index[0]
    dst = edge_index[1]
    pad = E_PAD - E
    src_p = jnp.concatenate([src, jnp.zeros((pad,), jnp.int32)]).reshape(NW, CPW, CHUNK)
    dst_p = jnp.concatenate([dst, jnp.full((pad,), N, jnp.int32)]).reshape(NW, CPW, CHUNK)
    zeros = jnp.zeros((N_SP, D), jnp.float32)

    params = [(W1_0, b1_0, W2_0, b2_0), (W1_1, b1_1, W2_1, b2_1),
              (W1_2, b1_2, W2_2, b2_2)]
    h = x
    for (w1, b1, w2, b2) in params:
        agg = _sc_agg(h, src_p, dst_p, zeros)
        h = _tc_mlp(h, agg, w1, b1.reshape(1, D), w2, b2.reshape(1, D))
    return h


# speedup vs baseline: 9.7754x; 3.0763x over previous
import functools

import jax
import jax.numpy as jnp
from jax import lax
from jax.experimental import pallas as pl
from jax.experimental.pallas import tpu as pltpu
from jax.experimental.pallas import tpu_sc as plsc

N = 10000
E = 320000
D = 128

NC = 2
NS = 16
NW = NC * NS

CHUNK = 128
CPW = 80
EPW = CPW * CHUNK
E_PAD = EPW * NW

ROWS_PER_TILE = 640
N_SP = ROWS_PER_TILE * NS

_sc_mesh = plsc.VectorSubcoreMesh(core_axis_name="c", subcore_axis_name="s")


@functools.partial(
    pl.kernel,
    out_type=jax.ShapeDtypeStruct((NC, N_SP, D), jnp.float32),
    mesh=_sc_mesh,
    scratch_types=[
        pltpu.VMEM((4, CHUNK), jnp.int32),
        pltpu.VMEM((4, CHUNK), jnp.int32),
        pltpu.VMEM((2, CHUNK, D), jnp.float32),
        pltpu.VMEM_SHARED((N_SP, D), jnp.float32),
        pltpu.SemaphoreType.DMA,
        pltpu.SemaphoreType.DMA,
        pltpu.SemaphoreType.DMA,
        pltpu.SemaphoreType.DMA,
        [pltpu.SemaphoreType.DMA] * 4,
    ],
)
def _sc_agg(h_hbm, src_hbm, dst_hbm, zeros_hbm, out_hbm,
            src_v, dst_v, rows_v, agg_sh, gsem0, gsem1, ssem0, ssem1, isems):
    cid = lax.axis_index("c")
    sid = lax.axis_index("s")
    wid = cid * NS + sid
    gsems = (gsem0, gsem1)
    ssems = (ssem0, ssem1)

    row0 = sid * ROWS_PER_TILE
    pltpu.sync_copy(zeros_hbm.at[pl.ds(row0, ROWS_PER_TILE)],
                    agg_sh.at[pl.ds(row0, ROWS_PER_TILE)])
    pltpu.sync_copy(src_hbm.at[wid, 0], src_v.at[0])
    pltpu.sync_copy(dst_hbm.at[wid, 0], dst_v.at[0])
    plsc.subcore_barrier()
    pltpu.async_copy(h_hbm.at[src_v.at[0]], rows_v.at[0], gsem0)
    for s in (1, 2):
        pltpu.async_copy(src_hbm.at[wid, s], src_v.at[s], isems[s])
        pltpu.async_copy(dst_hbm.at[wid, s], dst_v.at[s], isems[s])

    def _step(j, u):
        p, s = u % 2, u % 4
        q, s_1 = 1 - p, (u + 3) % 4
        s1 = (u + 1) % 4

        pltpu.make_async_copy(h_hbm.at[src_v.at[s]], rows_v.at[p],
                              gsems[p]).wait()

        @pl.when(j >= 1)
        def _():
            pltpu.make_async_copy(rows_v.at[q], agg_sh.at[dst_v.at[s_1]],
                                  ssems[q]).wait()

        pltpu.async_copy(rows_v.at[p], agg_sh.at[dst_v.at[s]], ssems[p],
                         add=True)

        @pl.when(j + 1 < CPW)
        def _():
            pltpu.make_async_copy(src_hbm.at[wid, j + 1], src_v.at[s1],
                                  isems[s1]).wait()
            pltpu.make_async_copy(dst_hbm.at[wid, j + 1], dst_v.at[s1],
                                  isems[s1]).wait()
            pltpu.async_copy(h_hbm.at[src_v.at[s1]], rows_v.at[q], gsems[q])

        @pl.when(j + 3 < CPW)
        def _():
            pltpu.async_copy(src_hbm.at[wid, j + 3], src_v.at[s_1], isems[s_1])
            pltpu.async_copy(dst_hbm.at[wid, j + 3], dst_v.at[s_1], isems[s_1])

    def body(i, carry):
        for u in range(4):
            _step(4 * i + u, u)
        return carry

    lax.fori_loop(0, CPW // 4, body, 0)
    pltpu.make_async_copy(rows_v.at[(CPW - 1) % 2],
                          agg_sh.at[dst_v.at[(CPW - 1) % 4]],
                          ssems[(CPW - 1) % 2]).wait()
    plsc.subcore_barrier()

    pltpu.sync_copy(agg_sh.at[pl.ds(row0, ROWS_PER_TILE)],
                    out_hbm.at[cid, pl.ds(row0, ROWS_PER_TILE)])


ROW_BLK = 1000


def _tc_mlp_body(h_ref, a0_ref, a1_ref, w1_ref, b1_ref, w2_ref, b2_ref, out_ref):
    m = h_ref[...] + a0_ref[0] + a1_ref[0]
    t = jnp.dot(m, w1_ref[...], preferred_element_type=jnp.float32) + b1_ref[...]
    t = jnp.maximum(t, 0.0)
    u = jnp.dot(t, w2_ref[...], preferred_element_type=jnp.float32) + b2_ref[...]
    out_ref[...] = jnp.where(u > 0.0, u, jnp.exp(jnp.minimum(u, 0.0)) - 1.0)


def _tc_mlp(h, agg, w1, b1, w2, b2):
    grid = N // ROW_BLK
    return pl.pallas_call(
        _tc_mlp_body,
        grid=(grid,),
        in_specs=[
            pl.BlockSpec((ROW_BLK, D), lambda i: (i, 0)),
            pl.BlockSpec((1, ROW_BLK, D), lambda i: (0, i, 0)),
            pl.BlockSpec((1, ROW_BLK, D), lambda i: (1, i, 0)),
            pl.BlockSpec((D, D), lambda i: (0, 0)),
            pl.BlockSpec((1, D), lambda i: (0, 0)),
            pl.BlockSpec((D, D), lambda i: (0, 0)),
            pl.BlockSpec((1, D), lambda i: (0, 0)),
        ],
        out_specs=pl.BlockSpec((ROW_BLK, D), lambda i: (i, 0)),
        out_shape=jax.ShapeDtypeStruct((N, D), jnp.float32),
    )(h, agg, agg, w1, b1, w2, b2)


def kernel(x, edge_index, W1_0, b1_0, W2_0, b2_0, W1_1, b1_1, W2_1, b2_1,
           W1_2, b1_2, W2_2, b2_2):
    src = edge_index[0]
    dst = edge_index[1]
    pad = E_PAD - E
    pad_iota = jnp.arange(pad, dtype=jnp.int32)
    pad_src = pad_iota % N
    pad_dst = N + pad_iota % (N_SP - N)
    src_p = jnp.concatenate([src, pad_src]).reshape(NW, CPW, CHUNK)
    dst_p = jnp.concatenate([dst, pad_dst]).reshape(NW, CPW, CHUNK)
    zeros = jnp.zeros((N_SP, D), jnp.float32)

    params = [(W1_0, b1_0, W2_0, b2_0), (W1_1, b1_1, W2_1, b2_1),
              (W1_2, b1_2, W2_2, b2_2)]
    h = x
    for (w1, b1, w2, b2) in params:
        agg = _sc_agg(h, src_p, dst_p, zeros)
        h = _tc_mlp(h, agg, w1, b1.reshape(1, D), w2, b2.reshape(1, D))
    return h


# speedup vs baseline: 12.5244x; 1.2812x over previous
import functools

import jax
import jax.numpy as jnp
from jax import lax
from jax.experimental import pallas as pl
from jax.experimental.pallas import tpu as pltpu
from jax.experimental.pallas import tpu_sc as plsc

N = 10000
E = 320000
D = 128

NC = 2
NS = 16
NW = NC * NS

CHUNK = 120
CPW = 84
EPW = CPW * CHUNK
E_PAD = EPW * NW

ROWS_PER_TILE = 632
N_SP = ROWS_PER_TILE * NS

_sc_mesh = plsc.VectorSubcoreMesh(core_axis_name="c", subcore_axis_name="s")


@functools.partial(
    pl.kernel,
    out_type=jax.ShapeDtypeStruct((NC, N_SP, D), jnp.float32),
    mesh=_sc_mesh,
    scratch_types=[
        pltpu.VMEM((3, CHUNK), jnp.int32),
        pltpu.VMEM((4, CHUNK), jnp.int32),
        pltpu.VMEM((3, CHUNK, D), jnp.float32),
        pltpu.VMEM_SHARED((N_SP, D), jnp.float32),
        [pltpu.SemaphoreType.DMA] * 3,
        [pltpu.SemaphoreType.DMA] * 2,
        [pltpu.SemaphoreType.DMA] * 3,
        [pltpu.SemaphoreType.DMA] * 4,
    ],
)
def _sc_agg(h_hbm, src_hbm, dst_hbm, zeros_hbm, out_hbm,
            src_v, dst_v, rows_v, agg_sh, gsems, ssems, isrc, idst):
    cid = lax.axis_index("c")
    sid = lax.axis_index("s")
    wid = cid * NS + sid

    row0 = sid * ROWS_PER_TILE
    pltpu.sync_copy(zeros_hbm.at[pl.ds(row0, ROWS_PER_TILE)],
                    agg_sh.at[pl.ds(row0, ROWS_PER_TILE)])
    pltpu.sync_copy(src_hbm.at[wid, 0], src_v.at[0])
    pltpu.sync_copy(dst_hbm.at[wid, 0], dst_v.at[0])
    plsc.subcore_barrier()
    pltpu.async_copy(h_hbm.at[src_v.at[0]], rows_v.at[0], gsems[0])
    for s in (1, 2):
        pltpu.async_copy(src_hbm.at[wid, s], src_v.at[s], isrc[s])
        pltpu.async_copy(dst_hbm.at[wid, s], dst_v.at[s], idst[s])
    pltpu.make_async_copy(src_hbm.at[wid, 1], src_v.at[1], isrc[1]).wait()
    pltpu.async_copy(h_hbm.at[src_v.at[1]], rows_v.at[1], gsems[1])

    def _step(j, u):
        r, d, p = u % 3, u % 4, u % 2
        r1, d1, p1 = (u + 2) % 3, (u + 3) % 4, 1 - (u % 2)

        pltpu.make_async_copy(h_hbm.at[src_v.at[r]], rows_v.at[r],
                              gsems[r]).wait()

        @pl.when(j >= 1)
        def _():
            pltpu.make_async_copy(rows_v.at[r1], agg_sh.at[dst_v.at[d1]],
                                  ssems[p1]).wait()

        @pl.when(j >= 1)
        def _():
            pltpu.make_async_copy(dst_hbm.at[wid, j], dst_v.at[d],
                                  idst[d]).wait()

        pltpu.async_copy(rows_v.at[r], agg_sh.at[dst_v.at[d]], ssems[p],
                         add=True)

        @pl.when(j + 3 < CPW)
        def _():
            pltpu.async_copy(src_hbm.at[wid, j + 3], src_v.at[r], isrc[r])
            pltpu.async_copy(dst_hbm.at[wid, j + 3], dst_v.at[d1], idst[d1])

        @pl.when(j + 2 < CPW)
        def _():
            pltpu.make_async_copy(src_hbm.at[wid, j + 2], src_v.at[r1],
                                  isrc[r1]).wait()
            pltpu.async_copy(h_hbm.at[src_v.at[r1]], rows_v.at[r1], gsems[r1])

    def body(i, carry):
        for u in range(12):
            _step(12 * i + u, u)
        return carry

    lax.fori_loop(0, CPW // 12, body, 0)
    pltpu.make_async_copy(rows_v.at[(CPW - 1) % 3],
                          agg_sh.at[dst_v.at[(CPW - 1) % 4]],
                          ssems[(CPW - 1) % 2]).wait()
    plsc.subcore_barrier()

    pltpu.sync_copy(agg_sh.at[pl.ds(row0, ROWS_PER_TILE)],
                    out_hbm.at[cid, pl.ds(row0, ROWS_PER_TILE)])


ROW_BLK = 1000


def _tc_mlp_body(h_ref, a0_ref, a1_ref, w1_ref, b1_ref, w2_ref, b2_ref, out_ref):
    m = h_ref[...] + a0_ref[0] + a1_ref[0]
    t = jnp.dot(m, w1_ref[...], preferred_element_type=jnp.float32) + b1_ref[...]
    t = jnp.maximum(t, 0.0)
    u = jnp.dot(t, w2_ref[...], preferred_element_type=jnp.float32) + b2_ref[...]
    out_ref[...] = jnp.where(u > 0.0, u, jnp.exp(jnp.minimum(u, 0.0)) - 1.0)


def _tc_mlp(h, agg, w1, b1, w2, b2):
    grid = N // ROW_BLK
    return pl.pallas_call(
        _tc_mlp_body,
        grid=(grid,),
        in_specs=[
            pl.BlockSpec((ROW_BLK, D), lambda i: (i, 0)),
            pl.BlockSpec((1, ROW_BLK, D), lambda i: (0, i, 0)),
            pl.BlockSpec((1, ROW_BLK, D), lambda i: (1, i, 0)),
            pl.BlockSpec((D, D), lambda i: (0, 0)),
            pl.BlockSpec((1, D), lambda i: (0, 0)),
            pl.BlockSpec((D, D), lambda i: (0, 0)),
            pl.BlockSpec((1, D), lambda i: (0, 0)),
        ],
        out_specs=pl.BlockSpec((ROW_BLK, D), lambda i: (i, 0)),
        out_shape=jax.ShapeDtypeStruct((N, D), jnp.float32),
    )(h, agg, agg, w1, b1, w2, b2)


def kernel(x, edge_index, W1_0, b1_0, W2_0, b2_0, W1_1, b1_1, W2_1, b2_1,
           W1_2, b1_2, W2_2, b2_2):
    src = edge_index[0]
    dst = edge_index[1]
    pad = E_PAD - E
    pad_iota = jnp.arange(pad, dtype=jnp.int32)
    pad_src = pad_iota % N
    pad_dst = N + pad_iota % (N_SP - N)
    src_p = jnp.concatenate([src, pad_src]).reshape(NW, CPW, CHUNK)
    dst_p = jnp.concatenate([dst, pad_dst]).reshape(NW, CPW, CHUNK)
    zeros = jnp.zeros((N_SP, D), jnp.float32)

    params = [(W1_0, b1_0, W2_0, b2_0), (W1_1, b1_1, W2_1, b2_1),
              (W1_2, b1_2, W2_2, b2_2)]
    h = x
    for (w1, b1, w2, b2) in params:
        agg = _sc_agg(h, src_p, dst_p, zeros)
        h = _tc_mlp(h, agg, w1, b1.reshape(1, D), w2, b2.reshape(1, D))
    return h


# speedup vs baseline: 12.8152x; 1.0232x over previous
import functools

import jax
import jax.numpy as jnp
from jax import lax
from jax.experimental import pallas as pl
from jax.experimental.pallas import tpu as pltpu
from jax.experimental.pallas import tpu_sc as plsc

N = 10000
E = 320000
D = 128

NC = 2
NS = 16
NW = NC * NS

CHUNK = 120
CPW = 84
EPW = CPW * CHUNK
E_PAD = EPW * NW

ROWS_PER_TILE = 632
N_SP = ROWS_PER_TILE * NS

_sc_mesh = plsc.VectorSubcoreMesh(core_axis_name="c", subcore_axis_name="s")


@functools.partial(
    pl.kernel,
    out_type=jax.ShapeDtypeStruct((NC, N_SP, D), jnp.float32),
    mesh=_sc_mesh,
    scratch_types=[
        pltpu.VMEM((3, CHUNK), jnp.int32),
        pltpu.VMEM((4, CHUNK), jnp.int32),
        pltpu.VMEM((3, CHUNK, D), jnp.float32),
        pltpu.VMEM_SHARED((N_SP, D), jnp.float32),
        [pltpu.SemaphoreType.DMA] * 3,
        [pltpu.SemaphoreType.DMA] * 2,
        [pltpu.SemaphoreType.DMA] * 3,
        [pltpu.SemaphoreType.DMA] * 4,
        pltpu.SemaphoreType.DMA,
    ],
)
def _sc_agg(h_hbm, src_hbm, dst_hbm, zeros_hbm, out_hbm,
            src_v, dst_v, rows_v, agg_sh, gsems, ssems, isrc, idst, zsem):
    cid = lax.axis_index("c")
    sid = lax.axis_index("s")
    wid = cid * NS + sid

    row0 = sid * ROWS_PER_TILE
    pltpu.async_copy(zeros_hbm.at[pl.ds(row0, ROWS_PER_TILE)],
                     agg_sh.at[pl.ds(row0, ROWS_PER_TILE)], zsem)
    for s in (0, 1, 2):
        pltpu.async_copy(src_hbm.at[wid, s], src_v.at[s], isrc[s])
        pltpu.async_copy(dst_hbm.at[wid, s], dst_v.at[s], idst[s])
    pltpu.make_async_copy(src_hbm.at[wid, 0], src_v.at[0], isrc[0]).wait()
    pltpu.async_copy(h_hbm.at[src_v.at[0]], rows_v.at[0], gsems[0])
    pltpu.make_async_copy(src_hbm.at[wid, 1], src_v.at[1], isrc[1]).wait()
    pltpu.async_copy(h_hbm.at[src_v.at[1]], rows_v.at[1], gsems[1])
    pltpu.make_async_copy(zeros_hbm.at[pl.ds(row0, ROWS_PER_TILE)],
                          agg_sh.at[pl.ds(row0, ROWS_PER_TILE)], zsem).wait()
    plsc.subcore_barrier()

    def _step(j, u):
        r, d, p = u % 3, u % 4, u % 2
        r1, d1, p1 = (u + 2) % 3, (u + 3) % 4, 1 - (u % 2)

        pltpu.make_async_copy(h_hbm.at[src_v.at[r]], rows_v.at[r],
                              gsems[r]).wait()

        @pl.when(j >= 1)
        def _():
            pltpu.make_async_copy(rows_v.at[r1], agg_sh.at[dst_v.at[d1]],
                                  ssems[p1]).wait()

        pltpu.make_async_copy(dst_hbm.at[wid, j], dst_v.at[d],
                              idst[d]).wait()

        pltpu.async_copy(rows_v.at[r], agg_sh.at[dst_v.at[d]], ssems[p],
                         add=True)

        @pl.when(j + 3 < CPW)
        def _():
            pltpu.async_copy(src_hbm.at[wid, j + 3], src_v.at[r], isrc[r])
            pltpu.async_copy(dst_hbm.at[wid, j + 3], dst_v.at[d1], idst[d1])

        @pl.when(j + 2 < CPW)
        def _():
            pltpu.make_async_copy(src_hbm.at[wid, j + 2], src_v.at[r1],
                                  isrc[r1]).wait()
            pltpu.async_copy(h_hbm.at[src_v.at[r1]], rows_v.at[r1], gsems[r1])

    def body(i, carry):
        for u in range(12):
            _step(12 * i + u, u)
        return carry

    lax.fori_loop(0, CPW // 12, body, 0)
    pltpu.make_async_copy(rows_v.at[(CPW - 1) % 3],
                          agg_sh.at[dst_v.at[(CPW - 1) % 4]],
                          ssems[(CPW - 1) % 2]).wait()
    plsc.subcore_barrier()

    pltpu.sync_copy(agg_sh.at[pl.ds(row0, ROWS_PER_TILE)],
                    out_hbm.at[cid, pl.ds(row0, ROWS_PER_TILE)])


ROW_BLK = 1000


def _tc_mlp_body(h_ref, a0_ref, a1_ref, w1_ref, b1_ref, w2_ref, b2_ref, out_ref):
    m = h_ref[...] + a0_ref[0] + a1_ref[0]
    t = jnp.dot(m, w1_ref[...], preferred_element_type=jnp.float32) + b1_ref[...]
    t = jnp.maximum(t, 0.0)
    u = jnp.dot(t, w2_ref[...], preferred_element_type=jnp.float32) + b2_ref[...]
    out_ref[...] = jnp.where(u > 0.0, u, jnp.exp(jnp.minimum(u, 0.0)) - 1.0)


def _tc_mlp(h, agg, w1, b1, w2, b2):
    grid = N // ROW_BLK
    return pl.pallas_call(
        _tc_mlp_body,
        grid=(grid,),
        in_specs=[
            pl.BlockSpec((ROW_BLK, D), lambda i: (i, 0)),
            pl.BlockSpec((1, ROW_BLK, D), lambda i: (0, i, 0)),
            pl.BlockSpec((1, ROW_BLK, D), lambda i: (1, i, 0)),
            pl.BlockSpec((D, D), lambda i: (0, 0)),
            pl.BlockSpec((1, D), lambda i: (0, 0)),
            pl.BlockSpec((D, D), lambda i: (0, 0)),
            pl.BlockSpec((1, D), lambda i: (0, 0)),
        ],
        out_specs=pl.BlockSpec((ROW_BLK, D), lambda i: (i, 0)),
        out_shape=jax.ShapeDtypeStruct((N, D), jnp.float32),
    )(h, agg, agg, w1, b1, w2, b2)


def kernel(x, edge_index, W1_0, b1_0, W2_0, b2_0, W1_1, b1_1, W2_1, b2_1,
           W1_2, b1_2, W2_2, b2_2):
    src = edge_index[0]
    dst = edge_index[1]
    pad = E_PAD - E
    pad_iota = jnp.arange(pad, dtype=jnp.int32)
    pad_src = pad_iota % N
    pad_dst = N + pad_iota % (N_SP - N)
    src_p = jnp.concatenate([src, pad_src]).reshape(NW, CPW, CHUNK)
    dst_p = jnp.concatenate([dst, pad_dst]).reshape(NW, CPW, CHUNK)
    zeros = jnp.zeros((N_SP, D), jnp.float32)

    params = [(W1_0, b1_0, W2_0, b2_0), (W1_1, b1_1, W2_1, b2_1),
              (W1_2, b1_2, W2_2, b2_2)]
    h = x
    for (w1, b1, w2, b2) in params:
        agg = _sc_agg(h, src_p, dst_p, zeros)
        h = _tc_mlp(h, agg, w1, b1.reshape(1, D), w2, b2.reshape(1, D))
    return h


# speedup vs baseline: 13.1578x; 1.0267x over previous
import functools

import jax
import jax.numpy as jnp
import numpy as np
from jax import lax
from jax.experimental import pallas as pl
from jax.experimental.pallas import tpu as pltpu
from jax.experimental.pallas import tpu_sc as plsc

N = 10000
E = 320000
D = 128

NC = 2
NS = 16
NW = NC * NS

CHUNK = 120
CPW = 84
EPW = CPW * CHUNK
E_PAD = EPW * NW

ROWS_PER_TILE = 632
N_SP = ROWS_PER_TILE * NS

_sc_mesh = plsc.VectorSubcoreMesh(core_axis_name="c", subcore_axis_name="s")


@functools.partial(
    pl.kernel,
    out_type=jax.ShapeDtypeStruct((NC, N_SP, D), jnp.float32),
    mesh=_sc_mesh,
    scratch_types=[
        pltpu.VMEM((3, CHUNK), jnp.int32),
        pltpu.VMEM((4, CHUNK), jnp.int32),
        pltpu.VMEM((3, CHUNK, D), jnp.float32),
        pltpu.VMEM_SHARED((N_SP, D), jnp.float32),
        [pltpu.SemaphoreType.DMA] * 3,
        [pltpu.SemaphoreType.DMA] * 2,
        [pltpu.SemaphoreType.DMA] * 3,
        [pltpu.SemaphoreType.DMA] * 4,
        pltpu.SemaphoreType.DMA,
    ],
)
def _sc_agg(h_hbm, src_hbm, dst_hbm, zeros_hbm, out_hbm,
            src_v, dst_v, rows_v, agg_sh, gsems, ssems, isrc, idst, zsem):
    cid = lax.axis_index("c")
    sid = lax.axis_index("s")
    wid = cid * NS + sid

    row0 = sid * ROWS_PER_TILE
    pltpu.async_copy(zeros_hbm.at[pl.ds(row0, ROWS_PER_TILE)],
                     agg_sh.at[pl.ds(row0, ROWS_PER_TILE)], zsem)
    for s in (0, 1, 2):
        pltpu.async_copy(src_hbm.at[wid, s], src_v.at[s], isrc[s])
        pltpu.async_copy(dst_hbm.at[wid, s], dst_v.at[s], idst[s])
    pltpu.make_async_copy(src_hbm.at[wid, 0], src_v.at[0], isrc[0]).wait()
    pltpu.async_copy(h_hbm.at[src_v.at[0]], rows_v.at[0], gsems[0])
    pltpu.make_async_copy(src_hbm.at[wid, 1], src_v.at[1], isrc[1]).wait()
    pltpu.async_copy(h_hbm.at[src_v.at[1]], rows_v.at[1], gsems[1])
    pltpu.make_async_copy(zeros_hbm.at[pl.ds(row0, ROWS_PER_TILE)],
                          agg_sh.at[pl.ds(row0, ROWS_PER_TILE)], zsem).wait()
    plsc.subcore_barrier()

    def _step(j, u):
        r, d, p = u % 3, u % 4, u % 2
        r1, d1, p1 = (u + 2) % 3, (u + 3) % 4, 1 - (u % 2)

        pltpu.make_async_copy(h_hbm.at[src_v.at[r]], rows_v.at[r],
                              gsems[r]).wait()

        @pl.when(j >= 1)
        def _():
            pltpu.make_async_copy(rows_v.at[r1], agg_sh.at[dst_v.at[d1]],
                                  ssems[p1]).wait()

        pltpu.make_async_copy(dst_hbm.at[wid, j], dst_v.at[d],
                              idst[d]).wait()

        pltpu.async_copy(rows_v.at[r], agg_sh.at[dst_v.at[d]], ssems[p],
                         add=True)

        @pl.when(j + 3 < CPW)
        def _():
            pltpu.async_copy(src_hbm.at[wid, j + 3], src_v.at[r], isrc[r])
            pltpu.async_copy(dst_hbm.at[wid, j + 3], dst_v.at[d1], idst[d1])

        @pl.when(j + 2 < CPW)
        def _():
            pltpu.make_async_copy(src_hbm.at[wid, j + 2], src_v.at[r1],
                                  isrc[r1]).wait()
            pltpu.async_copy(h_hbm.at[src_v.at[r1]], rows_v.at[r1], gsems[r1])

    def body(i, carry):
        for u in range(12):
            _step(12 * i + u, u)
        return carry

    lax.fori_loop(0, CPW // 12, body, 0)
    pltpu.make_async_copy(rows_v.at[(CPW - 1) % 3],
                          agg_sh.at[dst_v.at[(CPW - 1) % 4]],
                          ssems[(CPW - 1) % 2]).wait()
    plsc.subcore_barrier()

    pltpu.sync_copy(agg_sh.at[pl.ds(row0, ROWS_PER_TILE)],
                    out_hbm.at[cid, pl.ds(row0, ROWS_PER_TILE)])


ROW_BLK = 2000


def _tc_mlp_body(h_ref, a0_ref, a1_ref, w1_ref, b1_ref, w2_ref, b2_ref, out_ref):
    m = h_ref[...] + a0_ref[0] + a1_ref[0]
    t = jnp.dot(m, w1_ref[...], preferred_element_type=jnp.float32) + b1_ref[...]
    t = jnp.maximum(t, 0.0)
    u = jnp.dot(t, w2_ref[...], preferred_element_type=jnp.float32) + b2_ref[...]
    out_ref[...] = jnp.where(u > 0.0, u, jnp.exp(jnp.minimum(u, 0.0)) - 1.0)


def _tc_mlp(h, agg, w1, b1, w2, b2):
    grid = N // ROW_BLK
    return pl.pallas_call(
        _tc_mlp_body,
        grid=(grid,),
        in_specs=[
            pl.BlockSpec((ROW_BLK, D), lambda i: (i, 0)),
            pl.BlockSpec((1, ROW_BLK, D), lambda i: (0, i, 0)),
            pl.BlockSpec((1, ROW_BLK, D), lambda i: (1, i, 0)),
            pl.BlockSpec((D, D), lambda i: (0, 0)),
            pl.BlockSpec((1, D), lambda i: (0, 0)),
            pl.BlockSpec((D, D), lambda i: (0, 0)),
            pl.BlockSpec((1, D), lambda i: (0, 0)),
        ],
        out_specs=pl.BlockSpec((ROW_BLK, D), lambda i: (i, 0)),
        out_shape=jax.ShapeDtypeStruct((N, D), jnp.float32),
    )(h, agg, agg, w1, b1, w2, b2)


def kernel(x, edge_index, W1_0, b1_0, W2_0, b2_0, W1_1, b1_1, W2_1, b2_1,
           W1_2, b1_2, W2_2, b2_2):
    src = edge_index[0]
    dst = edge_index[1]
    pad = E_PAD - E
    pad_iota = np.arange(pad, dtype=np.int32)
    pad_src = jnp.asarray(pad_iota % N)
    pad_dst = jnp.asarray(N + pad_iota % (N_SP - N))
    src_p = jnp.concatenate([src, pad_src]).reshape(NW, CPW, CHUNK)
    dst_p = jnp.concatenate([dst, pad_dst]).reshape(NW, CPW, CHUNK)
    zeros = jnp.zeros((N_SP, D), jnp.float32)

    params = [(W1_0, b1_0, W2_0, b2_0), (W1_1, b1_1, W2_1, b2_1),
              (W1_2, b1_2, W2_2, b2_2)]
    h = x
    for (w1, b1, w2, b2) in params:
        agg = _sc_agg(h, src_p, dst_p, zeros)
        h = _tc_mlp(h, agg, w1, b1.reshape(1, D), w2, b2.reshape(1, D))
    return h
